# baseline (speedup 1.0000x reference)
import functools

import jax
import jax.numpy as jnp
from jax import lax
from jax.experimental import pallas as pl
from jax.experimental.pallas import tpu as pltpu
from jax.experimental.pallas import tpu_sc as plsc

D = 64
NROW = 1000000
NB = 16384
NS = 50
NW = 32

_mesh = plsc.VectorSubcoreMesh(core_axis_name="c", subcore_axis_name="s")

FW = 128
NBLK = NROW // FW
TAILC = NROW - NBLK * FW
NT_A = -(-NBLK // NW)


@functools.partial(
    pl.kernel,
    out_type=jax.ShapeDtypeStruct((NROW // 2, 128), jnp.float32),
    mesh=_mesh,
    compiler_params=pltpu.CompilerParams(use_tc_tiling_on_sc=True, needs_layout_passes=False),
    scratch_types=(
        [pltpu.VMEM((D, FW), jnp.float32)] * 2
        + [pltpu.VMEM((FW // 2, 128), jnp.float32)] * 2
        + [pltpu.VMEM((D, FW), jnp.float32)]
        + [pltpu.SemaphoreType.DMA] * 4
    ),
)
def _format_table(tt_hbm, tail_hbm, fmt_hbm, t0, t1, o0, o1, tl,
                  g0, g1, w0, w1):
    wid = lax.axis_index("s") * 2 + lax.axis_index("c")
    tb = (t0, t1)
    ob = (o0, o1)
    gsem = (g0, g1)
    wsem = (w0, w1)

    iota = lax.iota(jnp.int32, 16)
    half = iota // 2
    colpar = (iota % 2) * D

    def in_start(t, p):
        bid = wid + NW * t
        pltpu.make_async_copy(
            tt_hbm.at[:, pl.ds(bid * FW, FW)], tb[p], gsem[p]).start()

    def in_wait(p):
        pltpu.make_async_copy(
            tt_hbm.at[:, pl.ds(0, FW)], tb[p], gsem[p]).wait()

    def out_start(t, p):
        bid = wid + NW * t
        pltpu.make_async_copy(
            ob[p], fmt_hbm.at[pl.ds(bid * (FW // 2), FW // 2), :],
            wsem[p]).start()

    def out_wait(p):
        pltpu.make_async_copy(
            ob[p], fmt_hbm.at[pl.ds(0, FW // 2), :], wsem[p]).wait()

    def shuffle(src, dst, nchunk):
        @pl.loop(0, D)
        def _rows(d):
            colv = colpar + d
            for c in range(nchunk):
                v = src[d, pl.ds(16 * c, 16)]
                plsc.store_scatter(dst, [half + 8 * c, colv], v)

    in_start(0, 0)

    @pl.loop(0, NT_A, step=2)
    def _blocks(t0i):
        for b in range(2):
            t = t0i + b
            p = b
            bid = wid + NW * t

            @pl.when(bid < NBLK)
            def _():
                in_wait(p)

                @pl.when(bid + NW < NBLK)
                def _():
                    in_start(t + 1, 1 - p)

                @pl.when(t >= 2)
                def _():
                    out_wait(p)

                shuffle(tb[p], ob[p], FW // 16)
                out_start(t, p)

    out_wait((NT_A - 1) % 2)
    out_wait((NT_A - 2) % 2)

    @pl.when(wid == 0)
    def _():
        pltpu.sync_copy(tail_hbm, tl)
        shuffle(tl, ob[0], TAILC // 16)
        pltpu.sync_copy(ob[0].at[pl.ds(0, TAILC // 2), :],
                        fmt_hbm.at[pl.ds(NBLK * (FW // 2), TAILC // 2), :])


BPW = NB * NS // NW
NT_B = BPW // 128


@functools.partial(
    pl.kernel,
    out_type=jax.ShapeDtypeStruct((NS, 8, 128, 8, 128), jnp.float32),
    mesh=_mesh,
    compiler_params=pltpu.CompilerParams(use_tc_tiling_on_sc=False, needs_layout_passes=False),
    scratch_types=(
        [pltpu.VMEM((BPW,), jnp.int32)]
        + [pltpu.VMEM((128, D), jnp.float32)] * 2
        + [pltpu.VMEM((D, 128), jnp.float32)] * 2
        + [pltpu.SemaphoreType.DMA] * 4
    ),
)
def _gather_format(idx_hbm, tbl_hbm, out_hbm, idx_v, gb0, gb1, ob0, ob1,
                   g0, g1, w0, w1):
    wid = lax.axis_index("s") * 2 + lax.axis_index("c")
    base = wid * BPW
    gb = (gb0, gb1)
    ob = (ob0, ob1)
    gsem = (g0, g1)
    wsem = (w0, w1)

    iota = lax.iota(jnp.int32, 16)
    rowv = [iota + 16 * c for c in range(8)]
    colz = jnp.full((16,), 0, jnp.int32)

    pltpu.sync_copy(idx_hbm.at[pl.ds(base, BPW)], idx_v)

    def gather_start(t, p):
        pltpu.make_async_copy(
            tbl_hbm.at[idx_v.at[pl.ds(t * 128, 128)]], gb[p], gsem[p]).start()

    def gather_wait(p):
        pltpu.make_async_copy(
            tbl_hbm.at[idx_v.at[pl.ds(0, 128)]], gb[p], gsem[p]).wait()

    def write_start(t, p):
        k = wid * NT_B + t
        s = k // 128
        bt = k % 128
        for dt in range(8):
            pltpu.make_async_copy(
                ob[p].at[pl.ds(8 * dt, 8), :],
                out_hbm.at[s, dt, bt], wsem[p]).start()

    def write_wait(p):
        for dt in range(8):
            pltpu.make_async_copy(
                ob[p].at[pl.ds(8 * dt, 8), :],
                out_hbm.at[0, dt, 0], wsem[p]).wait()

    gather_start(0, 0)

    @pl.loop(0, NT_B, step=2)
    def _blocks(t0i):
        for b in range(2):
            t = t0i + b
            p = b
            gather_wait(p)

            @pl.when(t + 1 < NT_B)
            def _():
                gather_start(t + 1, 1 - p)

            @pl.when(t >= 2)
            def _():
                write_wait(p)

            for d in range(D):
                colv = colz + d
                for c in range(8):
                    v = plsc.load_gather(gb[p], [rowv[c], colv])
                    ob[p][d, pl.ds(16 * c, 16)] = v

            write_start(t, p)

    write_wait((NT_B - 1) % 2)
    write_wait((NT_B - 2) % 2)


def kernel(x, table):
    flat = x.T.reshape(-1).astype(jnp.int32)
    out5 = _gather_format(flat, table)
    return out5.transpose(2, 4, 0, 1, 3).reshape(NB, NS, D)


# speedup vs baseline: 1.0055x; 1.0055x over previous
import functools

import jax
import jax.numpy as jnp
from jax import lax
from jax.experimental import pallas as pl
from jax.experimental.pallas import tpu as pltpu
from jax.experimental.pallas import tpu_sc as plsc

D = 64
NROW = 1000000
NB = 16384
NS = 50
NW = 32

_mesh = plsc.VectorSubcoreMesh(core_axis_name="c", subcore_axis_name="s")

FW = 128
NBLK = NROW // FW
TAILC = NROW - NBLK * FW
NT_A = -(-NBLK // NW)


@functools.partial(
    pl.kernel,
    out_type=jax.ShapeDtypeStruct((NROW // 2, 128), jnp.float32),
    mesh=_mesh,
    compiler_params=pltpu.CompilerParams(use_tc_tiling_on_sc=True, needs_layout_passes=False, disable_bounds_checks=True),
    scratch_types=(
        [pltpu.VMEM((D, FW), jnp.float32)] * 2
        + [pltpu.VMEM((FW // 2, 128), jnp.float32)] * 2
        + [pltpu.VMEM((D, FW), jnp.float32)]
        + [pltpu.SemaphoreType.DMA] * 4
    ),
)
def _format_table(tt_hbm, tail_hbm, fmt_hbm, t0, t1, o0, o1, tl,
                  g0, g1, w0, w1):
    wid = lax.axis_index("s") * 2 + lax.axis_index("c")
    tb = (t0, t1)
    ob = (o0, o1)
    gsem = (g0, g1)
    wsem = (w0, w1)

    iota = lax.iota(jnp.int32, 16)
    half = iota // 2
    colpar = (iota % 2) * D

    def in_start(t, p):
        bid = wid + NW * t
        pltpu.make_async_copy(
            tt_hbm.at[:, pl.ds(bid * FW, FW)], tb[p], gsem[p]).start()

    def in_wait(p):
        pltpu.make_async_copy(
            tt_hbm.at[:, pl.ds(0, FW)], tb[p], gsem[p]).wait()

    def out_start(t, p):
        bid = wid + NW * t
        pltpu.make_async_copy(
            ob[p], fmt_hbm.at[pl.ds(bid * (FW // 2), FW // 2), :],
            wsem[p]).start()

    def out_wait(p):
        pltpu.make_async_copy(
            ob[p], fmt_hbm.at[pl.ds(0, FW // 2), :], wsem[p]).wait()

    def shuffle(src, dst, nchunk):
        @pl.loop(0, D)
        def _rows(d):
            colv = colpar + d
            for c in range(nchunk):
                v = src[d, pl.ds(16 * c, 16)]
                plsc.store_scatter(dst, [half + 8 * c, colv], v)

    in_start(0, 0)

    @pl.loop(0, NT_A, step=2)
    def _blocks(t0i):
        for b in range(2):
            t = t0i + b
            p = b
            bid = wid + NW * t

            @pl.when(bid < NBLK)
            def _():
                in_wait(p)

                @pl.when(bid + NW < NBLK)
                def _():
                    in_start(t + 1, 1 - p)

                @pl.when(t >= 2)
                def _():
                    out_wait(p)

                shuffle(tb[p], ob[p], FW // 16)
                out_start(t, p)

    out_wait((NT_A - 1) % 2)
    out_wait((NT_A - 2) % 2)

    @pl.when(wid == 0)
    def _():
        pltpu.sync_copy(tail_hbm, tl)
        shuffle(tl, ob[0], TAILC // 16)
        pltpu.sync_copy(ob[0].at[pl.ds(0, TAILC // 2), :],
                        fmt_hbm.at[pl.ds(NBLK * (FW // 2), TAILC // 2), :])


BPW = NB * NS // NW
NT_B = BPW // 128


@functools.partial(
    pl.kernel,
    out_type=jax.ShapeDtypeStruct((NS, 8, 128, 8, 128), jnp.float32),
    mesh=_mesh,
    compiler_params=pltpu.CompilerParams(use_tc_tiling_on_sc=False, needs_layout_passes=False, disable_bounds_checks=True),
    scratch_types=(
        [pltpu.VMEM((BPW,), jnp.int32)]
        + [pltpu.VMEM((128, D), jnp.float32)] * 2
        + [pltpu.VMEM((D, 128), jnp.float32)] * 2
        + [pltpu.SemaphoreType.DMA] * 4
    ),
)
def _gather_format(idx_hbm, tbl_hbm, out_hbm, idx_v, gb0, gb1, ob0, ob1,
                   g0, g1, w0, w1):
    wid = lax.axis_index("s") * 2 + lax.axis_index("c")
    base = wid * BPW
    gb = (gb0, gb1)
    ob = (ob0, ob1)
    gsem = (g0, g1)
    wsem = (w0, w1)

    iota = lax.iota(jnp.int32, 16)
    rowv = [iota + 16 * c for c in range(8)]
    colz = jnp.full((16,), 0, jnp.int32)

    pltpu.sync_copy(idx_hbm.at[pl.ds(base, BPW)], idx_v)

    def gather_start(t, p):
        pltpu.make_async_copy(
            tbl_hbm.at[idx_v.at[pl.ds(t * 128, 128)]], gb[p], gsem[p]).start()

    def gather_wait(p):
        pltpu.make_async_copy(
            tbl_hbm.at[idx_v.at[pl.ds(0, 128)]], gb[p], gsem[p]).wait()

    def write_start(t, p):
        k = wid * NT_B + t
        s = k // 128
        bt = k % 128
        for dt in range(8):
            pltpu.make_async_copy(
                ob[p].at[pl.ds(8 * dt, 8), :],
                out_hbm.at[s, dt, bt], wsem[p]).start()

    def write_wait(p):
        for dt in range(8):
            pltpu.make_async_copy(
                ob[p].at[pl.ds(8 * dt, 8), :],
                out_hbm.at[0, dt, 0], wsem[p]).wait()

    gather_start(0, 0)

    @pl.loop(0, NT_B, step=2)
    def _blocks(t0i):
        for b in range(2):
            t = t0i + b
            p = b
            gather_wait(p)

            @pl.when(t + 1 < NT_B)
            def _():
                gather_start(t + 1, 1 - p)

            @pl.when(t >= 2)
            def _():
                write_wait(p)

            for d in range(D):
                colv = colz + d
                for c in range(8):
                    v = plsc.load_gather(gb[p], [rowv[c], colv])
                    ob[p][d, pl.ds(16 * c, 16)] = v

            write_start(t, p)

    write_wait((NT_B - 1) % 2)
    write_wait((NT_B - 2) % 2)


def kernel(x, table):
    flat = x.T.reshape(-1).astype(jnp.int32)
    out5 = _gather_format(flat, table)
    return out5.transpose(2, 4, 0, 1, 3).reshape(NB, NS, D)


# speedup vs baseline: 1.6749x; 1.6657x over previous
import functools

import jax
import jax.numpy as jnp
from jax import lax
from jax.experimental import pallas as pl
from jax.experimental.pallas import tpu as pltpu
from jax.experimental.pallas import tpu_sc as plsc

D = 64
B = 16384 * 50
NW = 32
BPW = B // NW
C = 400
NBUF = 4
NCH = BPW // C

_mesh = plsc.VectorSubcoreMesh(core_axis_name="c", subcore_axis_name="s")


@functools.partial(
    pl.kernel,
    out_type=jax.ShapeDtypeStruct((B, D), jnp.float32),
    mesh=_mesh,
    compiler_params=pltpu.CompilerParams(use_tc_tiling_on_sc=False),
    scratch_types=(
        [pltpu.VMEM((BPW,), jnp.int32)]
        + [pltpu.VMEM((C, D), jnp.float32)] * NBUF
        + [pltpu.SemaphoreType.DMA] * NBUF
        + [pltpu.SemaphoreType.DMA] * NBUF
    ),
)
def _embedding_gather(idx_hbm, table_hbm, out_hbm, idx_v, *bufs):
    rows = bufs[:NBUF]
    gsem = bufs[NBUF:2 * NBUF]
    wsem = bufs[2 * NBUF:]
    wid = lax.axis_index("s") * 2 + lax.axis_index("c")
    base = wid * BPW

    pltpu.sync_copy(idx_hbm.at[pl.ds(base, BPW)], idx_v)

    def gather_start(j, p):
        pltpu.make_async_copy(
            table_hbm.at[idx_v.at[pl.ds(j * C, C)]], rows[p], gsem[p]
        ).start()

    def gather_wait(p):
        pltpu.make_async_copy(
            table_hbm.at[idx_v.at[pl.ds(0, C)]], rows[p], gsem[p]
        ).wait()

    def write_start(j, p):
        pltpu.make_async_copy(
            rows[p], out_hbm.at[pl.ds(base + j * C, C)], wsem[p]
        ).start()

    def write_wait(p):
        pltpu.make_async_copy(
            rows[p], out_hbm.at[pl.ds(base, C)], wsem[p]
        ).wait()

    for j in range(NBUF - 1):
        gather_start(j, j)

    @pl.loop(0, NCH, step=NBUF)
    def _chunks(j0):
        for b in range(NBUF):
            j = j0 + b
            p = b
            gather_wait(p)

            q = (b - 1) % NBUF

            @pl.when(j >= 1)
            def _():
                write_wait(q)

            @pl.when(j + NBUF - 1 < NCH)
            def _():
                gather_start(j + NBUF - 1, q)

            write_start(j, p)

    write_wait((NCH - 1) % NBUF)


def kernel(x, table):
    n, s = x.shape
    flat = x.T.reshape(-1).astype(jnp.int32)
    out = _embedding_gather(flat, table)
    return out.reshape(s, n, D).transpose(1, 0, 2)
